# baseline (speedup 1.0000x reference)
import functools

import jax
import jax.numpy as jnp
from jax import lax
from jax.experimental import pallas as pl
from jax.experimental.pallas import tpu as pltpu
from jax.experimental.pallas import tpu_sc as plsc

D_MODEL = 768
D_FF = 3072
E = 8
TEMP = 0.7
N = 2048
NA = 2 * N

TM = 512
NB = NA // TM
TPAD = 24

NC, NS = 2, 16
NW = NC * NS
SPW = N // NW
CPW = N // NW


def _rinv_body(x_ref, wr_ref, br_ref,
               wk_ref, posi_ref, meta_ref, aux_ref, tok_ref, ws_ref):
    b = pl.program_id(0)

    @pl.when(b == 0)
    def _router_step():
        x = x_ref[...]
        logits = jnp.dot(x, wr_ref[...], preferred_element_type=jnp.float32)
        logits = logits + br_ref[...]
        z = logits / TEMP
        z = z - jnp.max(z, axis=1, keepdims=True)
        ez = jnp.exp(z)
        probs = ez / jnp.sum(ez, axis=1, keepdims=True)

        e_ids = lax.broadcasted_iota(jnp.int32, (N, E), 1)
        m1 = jnp.max(probs, axis=1, keepdims=True)
        i1 = jnp.min(jnp.where(probs == m1, e_ids, E), axis=1, keepdims=True)
        oh1 = (e_ids == i1).astype(jnp.float32)
        p2 = jnp.where(oh1 > 0, -jnp.inf, probs)
        m2 = jnp.max(p2, axis=1, keepdims=True)
        i2 = jnp.min(jnp.where(p2 == m2, e_ids, E), axis=1, keepdims=True)
        oh2 = (e_ids == i2).astype(jnp.float32)

        denom = m1 + m2 + 1e-6
        w1n = m1 / denom
        w2n = m2 / denom

        cum = oh1 + oh2
        k = 1
        while k < N:
            cum = cum + jnp.concatenate(
                [jnp.zeros((k, E), jnp.float32), cum[:N - k]], axis=0)
            k *= 2

        counts = cum[N - 1:N, :]
        offs = jnp.zeros_like(counts)
        for k in range(1, E):
            offs = offs + jnp.concatenate(
                [jnp.zeros((1, k), jnp.float32), counts[:, :E - k]], axis=1)

        slot = offs + cum - 1.0
        pos0 = jnp.sum(oh1 * slot, axis=1, keepdims=True)
        pos1 = jnp.sum(oh2 * slot, axis=1, keepdims=True)

        wk_ref[...] = jnp.concatenate([w1n, w2n], axis=1)
        posi_ref[...] = jnp.concatenate([pos0, pos1],
                                        axis=1).astype(jnp.int32)

        f_i = counts / N
        p_i = jnp.mean(probs, axis=0, keepdims=True)
        aux_ref[...] = E * jnp.sum(f_i * p_i, keepdims=True).reshape(1, 1)

        starts = offs
        ends = offs + counts
        r_col = lax.broadcasted_iota(jnp.int32, (NB, 1), 0)
        r_colf = r_col.astype(jnp.float32)
        flag = ((starts < (r_colf + 1.0) * TM) & (ends > r_colf * TM)
                & (counts > 0))
        flag_f = flag.astype(jnp.float32)
        within = flag_f
        k = 1
        while k < E:
            within = within + jnp.concatenate(
                [jnp.zeros((NB, k), jnp.float32), within[:, :E - k]], axis=1)
            k *= 2
        rowsum = jnp.sum(flag_f, axis=1, keepdims=True)
        prev = rowsum
        k = 1
        while k < NB:
            prev = prev + jnp.concatenate(
                [jnp.zeros((k, 1), jnp.float32), prev[:NB - k]], axis=0)
            k *= 2
        prev = prev - rowsum
        tidx = prev + within - 1.0
        lo = jnp.clip(starts - r_colf * TM, 0.0, float(TM))
        hi = jnp.clip(ends - r_colf * TM, 0.0, float(TM))

        trange = lax.broadcasted_iota(jnp.int32, (1, TPAD),
                                      1).astype(jnp.float32)
        me = jnp.zeros((1, TPAD), jnp.float32)
        mr = jnp.zeros((1, TPAD), jnp.float32)
        mlo = jnp.zeros((1, TPAD), jnp.float32)
        mhi = jnp.zeros((1, TPAD), jnp.float32)
        cov = jnp.zeros((1, TPAD), jnp.float32)
        for e in range(E):
            mf = ((tidx[:, e:e + 1] == trange)
                  & flag[:, e:e + 1]).astype(jnp.float32)
            me = me + jnp.sum(mf, axis=0, keepdims=True) * e
            mr = mr + jnp.sum(mf * r_colf, axis=0, keepdims=True)
            mlo = mlo + jnp.sum(mf * lo[:, e:e + 1], axis=0, keepdims=True)
            mhi = mhi + jnp.sum(mf * hi[:, e:e + 1], axis=0, keepdims=True)
            cov = cov + jnp.sum(mf, axis=0, keepdims=True)
        mr = mr + (1.0 - cov) * (NB - 1)
        meta_ref[...] = jnp.concatenate([me, mr, mlo, mhi],
                                        axis=0).astype(jnp.int32)

    @pl.when(b > 0)
    def _invert_step():
        slot = (lax.broadcasted_iota(jnp.int32, (1, _INV_B), 1)
                + _INV_B * (b - 1))
        posi = posi_ref[...]
        wkv = wk_ref[...]
        m0 = (posi[:, 0:1] == slot).astype(jnp.float32)
        m1 = (posi[:, 1:2] == slot).astype(jnp.float32)
        t_ids = lax.broadcasted_iota(jnp.int32, (N, 1),
                                     0).astype(jnp.float32)
        tok = jnp.sum((m0 + m1) * t_ids, axis=0, keepdims=True)
        w = jnp.sum(m0 * wkv[:, 0:1] + m1 * wkv[:, 1:2],
                    axis=0, keepdims=True)
        tok_ref[0] = tok.astype(jnp.int32)
        ws_ref[0] = w


_INV_B = 1024
_NBLK = NA // _INV_B


def _router_inv(x_flat, Wr, br):
    return pl.pallas_call(
        _rinv_body,
        grid=(1 + _NBLK,),
        in_specs=[
            pl.BlockSpec((N, D_MODEL), lambda b: (0, 0)),
            pl.BlockSpec((D_MODEL, E), lambda b: (0, 0)),
            pl.BlockSpec((1, E), lambda b: (0, 0)),
        ],
        out_specs=[
            pl.BlockSpec((N, 2), lambda b: (0, 0)),
            pl.BlockSpec((N, 2), lambda b: (0, 0)),
            pl.BlockSpec((4, TPAD), lambda b: (0, 0)),
            pl.BlockSpec((1, 1), lambda b: (0, 0)),
            pl.BlockSpec((1, 1, _INV_B),
                         lambda b: (jnp.maximum(b - 1, 0), 0, 0)),
            pl.BlockSpec((1, 1, _INV_B),
                         lambda b: (jnp.maximum(b - 1, 0), 0, 0)),
        ],
        out_shape=[
            jax.ShapeDtypeStruct((N, 2), jnp.float32),
            jax.ShapeDtypeStruct((N, 2), jnp.int32),
            jax.ShapeDtypeStruct((4, TPAD), jnp.int32),
            jax.ShapeDtypeStruct((1, 1), jnp.float32),
            jax.ShapeDtypeStruct((_NBLK, 1, _INV_B), jnp.int32),
            jax.ShapeDtypeStruct((_NBLK, 1, _INV_B), jnp.float32),
        ],
    )(x_flat, Wr, br.reshape(1, E))


@functools.lru_cache(maxsize=None)
def _sc_mesh():
    return plsc.VectorSubcoreMesh(
        core_axis_name="c", subcore_axis_name="s",
        num_cores=NC, num_subcores=NS)


GPW = NA // NW


def _sc_gather(tok_sorted, x_flat):
    @functools.partial(
        pl.kernel,
        out_type=jax.ShapeDtypeStruct((NA, D_MODEL), jnp.float32),
        mesh=_sc_mesh(),
        scratch_types=[
            pltpu.VMEM((GPW,), jnp.int32),
            pltpu.VMEM((GPW, D_MODEL), jnp.float32),
            pltpu.SemaphoreType.DMA,
        ],
    )
    def body(tok_hbm, x_hbm, out_hbm, idx_v, rows_v, sem):
        wid = lax.axis_index("s") * NC + lax.axis_index("c")
        base = wid * GPW
        pltpu.sync_copy(tok_hbm.at[pl.ds(base, GPW)], idx_v)
        pltpu.async_copy(x_hbm.at[idx_v], rows_v, sem).wait()
        pltpu.sync_copy(rows_v, out_hbm.at[pl.ds(base, GPW)])

    return body(tok_sorted, x_flat)


def _ffn_body(meta_ref, x_ref, w1_ref, b1_ref, w2_ref, b2_ref, ws_ref,
              out_ref):
    t = pl.program_id(0)
    r = meta_ref[1, t]
    r_prev = meta_ref[1, jnp.maximum(t - 1, 0)]
    first = (t == 0) | (r != r_prev)
    lo = meta_ref[2, t]
    hi = meta_ref[3, t]

    rows = lax.broadcasted_iota(jnp.int32, (TM, 1), 0)
    valid = (rows >= lo) & (rows < hi)

    x = x_ref[...]
    h = jnp.dot(x, w1_ref[0], preferred_element_type=jnp.float32)
    h = h + b1_ref[0]
    h = h * (1.0 / (1.0 + jnp.exp(-h)))
    contrib = jnp.dot(h, w2_ref[0], preferred_element_type=jnp.float32)
    contrib = contrib + b2_ref[0]
    update = jnp.where(valid, ws_ref[...] * contrib, 0.0)

    @pl.when(first)
    def _():
        out_ref[...] = update

    @pl.when(jnp.logical_not(first))
    def _():
        out_ref[...] += update


def _ffn_grouped(meta, x_sorted, W1, b1, W2, b2, w_sorted):
    grid_spec = pltpu.PrefetchScalarGridSpec(
        num_scalar_prefetch=1,
        grid=(TPAD,),
        in_specs=[
            pl.BlockSpec((TM, D_MODEL), lambda t, m: (m[1, t], 0)),
            pl.BlockSpec((1, D_MODEL, D_FF), lambda t, m: (m[0, t], 0, 0)),
            pl.BlockSpec((1, 1, D_FF), lambda t, m: (m[0, t], 0, 0)),
            pl.BlockSpec((1, D_FF, D_MODEL), lambda t, m: (m[0, t], 0, 0)),
            pl.BlockSpec((1, 1, D_MODEL), lambda t, m: (m[0, t], 0, 0)),
            pl.BlockSpec((TM, 1), lambda t, m: (m[1, t], 0)),
        ],
        out_specs=pl.BlockSpec((TM, D_MODEL), lambda t, m: (m[1, t], 0)),
    )
    return pl.pallas_call(
        _ffn_body,
        grid_spec=grid_spec,
        out_shape=jax.ShapeDtypeStruct((NA, D_MODEL), jnp.float32),
    )(meta, x_sorted, W1, b1.reshape(E, 1, D_FF), W2,
      b2.reshape(E, 1, D_MODEL), w_sorted)


def _sc_combine(i0, i1, out_sorted):
    @functools.partial(
        pl.kernel,
        out_type=jax.ShapeDtypeStruct((N, D_MODEL), jnp.float32),
        mesh=_sc_mesh(),
        scratch_types=[
            pltpu.VMEM((CPW,), jnp.int32),
            pltpu.VMEM((CPW,), jnp.int32),
            pltpu.VMEM((CPW, D_MODEL), jnp.float32),
            pltpu.VMEM((CPW, D_MODEL), jnp.float32),
            pltpu.SemaphoreType.DMA,
        ],
    )
    def body(i0_hbm, i1_hbm, os_hbm, out_hbm, idx0, idx1, r0, r1, sem):
        wid = lax.axis_index("s") * NC + lax.axis_index("c")
        base = wid * CPW
        pltpu.sync_copy(i0_hbm.at[pl.ds(base, CPW)], idx0)
        pltpu.sync_copy(i1_hbm.at[pl.ds(base, CPW)], idx1)
        pltpu.async_copy(os_hbm.at[idx0], r0, sem).wait()
        pltpu.async_copy(os_hbm.at[idx1], r1, sem).wait()

        def add_row(i, _):
            for j in range(D_MODEL // 16):
                sl = pl.ds(j * 16, 16)
                r0[i, sl] = r0[i, sl] + r1[i, sl]
            return 0

        lax.fori_loop(0, CPW, add_row, 0)
        pltpu.sync_copy(r0, out_hbm.at[pl.ds(base, CPW)])

    return body(i0, i1, out_sorted)


@jax.jit
def kernel(x, Wr, br, W1, b1, W2, b2):
    B, L, D = x.shape
    x_flat = x.reshape(-1, D)
    wk, posi, meta, aux, tok3, ws3 = _router_inv(x_flat, Wr, br)
    tok_sorted = tok3.reshape(-1)
    w_sorted = ws3.reshape(-1, 1)
    i0 = posi[:, 0]
    i1 = posi[:, 1]
    x_sorted = _sc_gather(tok_sorted, x_flat)
    out_sorted = _ffn_grouped(meta, x_sorted, W1, b1, W2, b2, w_sorted)
    out = _sc_combine(i0, i1, out_sorted)
    return out.reshape(B, L, D), aux.reshape(1)


# speedup vs baseline: 1.1554x; 1.1554x over previous
import functools

import jax
import jax.numpy as jnp
from jax import lax
from jax.experimental import pallas as pl
from jax.experimental.pallas import tpu as pltpu
from jax.experimental.pallas import tpu_sc as plsc

D_MODEL = 768
D_FF = 3072
E = 8
TEMP = 0.7
N = 2048
NA = 2 * N

TM = 128
NB = NA // TM
TPAD = 40

NC, NS = 2, 16
NW = NC * NS
SPW = N // NW
CPW = N // NW


def _rinv_body(x_ref, wr_ref, br_ref,
               wk_ref, posi_ref, meta_ref, aux_ref, tok_ref, ws_ref):
    b = pl.program_id(0)

    @pl.when(b == 0)
    def _router_step():
        x = x_ref[...]
        logits = jnp.dot(x, wr_ref[...], preferred_element_type=jnp.float32)
        logits = logits + br_ref[...]
        z = logits / TEMP
        z = z - jnp.max(z, axis=1, keepdims=True)
        ez = jnp.exp(z)
        probs = ez / jnp.sum(ez, axis=1, keepdims=True)

        e_ids = lax.broadcasted_iota(jnp.int32, (N, E), 1)
        m1 = jnp.max(probs, axis=1, keepdims=True)
        i1 = jnp.min(jnp.where(probs == m1, e_ids, E), axis=1, keepdims=True)
        oh1 = (e_ids == i1).astype(jnp.float32)
        p2 = jnp.where(oh1 > 0, -jnp.inf, probs)
        m2 = jnp.max(p2, axis=1, keepdims=True)
        i2 = jnp.min(jnp.where(p2 == m2, e_ids, E), axis=1, keepdims=True)
        oh2 = (e_ids == i2).astype(jnp.float32)

        denom = m1 + m2 + 1e-6
        w1n = m1 / denom
        w2n = m2 / denom

        cum = oh1 + oh2
        k = 1
        while k < N:
            cum = cum + jnp.concatenate(
                [jnp.zeros((k, E), jnp.float32), cum[:N - k]], axis=0)
            k *= 2

        counts = cum[N - 1:N, :]
        offs = jnp.zeros_like(counts)
        for k in range(1, E):
            offs = offs + jnp.concatenate(
                [jnp.zeros((1, k), jnp.float32), counts[:, :E - k]], axis=1)

        slot = offs + cum - 1.0
        pos0 = jnp.sum(oh1 * slot, axis=1, keepdims=True)
        pos1 = jnp.sum(oh2 * slot, axis=1, keepdims=True)

        wk_ref[...] = jnp.concatenate([w1n, w2n], axis=1)
        posi_ref[...] = jnp.concatenate([pos0, pos1],
                                        axis=1).astype(jnp.int32)

        f_i = counts / N
        p_i = jnp.mean(probs, axis=0, keepdims=True)
        aux_ref[...] = E * jnp.sum(f_i * p_i, keepdims=True).reshape(1, 1)

        starts = offs
        ends = offs + counts
        r_col = lax.broadcasted_iota(jnp.int32, (NB, 1), 0)
        r_colf = r_col.astype(jnp.float32)
        flag = ((starts < (r_colf + 1.0) * TM) & (ends > r_colf * TM)
                & (counts > 0))
        flag_f = flag.astype(jnp.float32)
        within = flag_f
        k = 1
        while k < E:
            within = within + jnp.concatenate(
                [jnp.zeros((NB, k), jnp.float32), within[:, :E - k]], axis=1)
            k *= 2
        rowsum = jnp.sum(flag_f, axis=1, keepdims=True)
        prev = rowsum
        k = 1
        while k < NB:
            prev = prev + jnp.concatenate(
                [jnp.zeros((k, 1), jnp.float32), prev[:NB - k]], axis=0)
            k *= 2
        prev = prev - rowsum
        tidx = prev + within - 1.0
        lo = jnp.clip(starts - r_colf * TM, 0.0, float(TM))
        hi = jnp.clip(ends - r_colf * TM, 0.0, float(TM))

        trange = lax.broadcasted_iota(jnp.int32, (1, TPAD),
                                      1).astype(jnp.float32)
        me = jnp.zeros((1, TPAD), jnp.float32)
        mr = jnp.zeros((1, TPAD), jnp.float32)
        mlo = jnp.zeros((1, TPAD), jnp.float32)
        mhi = jnp.zeros((1, TPAD), jnp.float32)
        cov = jnp.zeros((1, TPAD), jnp.float32)
        for e in range(E):
            mf = ((tidx[:, e:e + 1] == trange)
                  & flag[:, e:e + 1]).astype(jnp.float32)
            me = me + jnp.sum(mf, axis=0, keepdims=True) * e
            mr = mr + jnp.sum(mf * r_colf, axis=0, keepdims=True)
            mlo = mlo + jnp.sum(mf * lo[:, e:e + 1], axis=0, keepdims=True)
            mhi = mhi + jnp.sum(mf * hi[:, e:e + 1], axis=0, keepdims=True)
            cov = cov + jnp.sum(mf, axis=0, keepdims=True)
        mr = mr + (1.0 - cov) * (NB - 1)
        meta_ref[...] = jnp.concatenate([me, mr, mlo, mhi],
                                        axis=0).astype(jnp.int32)

    @pl.when(b > 0)
    def _invert_step():
        slot = (lax.broadcasted_iota(jnp.int32, (1, _INV_B), 1)
                + _INV_B * (b - 1))
        posi = posi_ref[...]
        wkv = wk_ref[...]
        m0 = (posi[:, 0:1] == slot).astype(jnp.float32)
        m1 = (posi[:, 1:2] == slot).astype(jnp.float32)
        t_ids = lax.broadcasted_iota(jnp.int32, (N, 1),
                                     0).astype(jnp.float32)
        tok = jnp.sum((m0 + m1) * t_ids, axis=0, keepdims=True)
        w = jnp.sum(m0 * wkv[:, 0:1] + m1 * wkv[:, 1:2],
                    axis=0, keepdims=True)
        tok_ref[0] = tok.astype(jnp.int32)
        ws_ref[0] = w


_INV_B = 1024
_NBLK = NA // _INV_B


def _router_inv(x_flat, Wr, br):
    return pl.pallas_call(
        _rinv_body,
        grid=(1 + _NBLK,),
        in_specs=[
            pl.BlockSpec((N, D_MODEL), lambda b: (0, 0)),
            pl.BlockSpec((D_MODEL, E), lambda b: (0, 0)),
            pl.BlockSpec((1, E), lambda b: (0, 0)),
        ],
        out_specs=[
            pl.BlockSpec((N, 2), lambda b: (0, 0)),
            pl.BlockSpec((N, 2), lambda b: (0, 0)),
            pl.BlockSpec((4, TPAD), lambda b: (0, 0)),
            pl.BlockSpec((1, 1), lambda b: (0, 0)),
            pl.BlockSpec((1, 1, _INV_B),
                         lambda b: (jnp.maximum(b - 1, 0), 0, 0)),
            pl.BlockSpec((1, 1, _INV_B),
                         lambda b: (jnp.maximum(b - 1, 0), 0, 0)),
        ],
        out_shape=[
            jax.ShapeDtypeStruct((N, 2), jnp.float32),
            jax.ShapeDtypeStruct((N, 2), jnp.int32),
            jax.ShapeDtypeStruct((4, TPAD), jnp.int32),
            jax.ShapeDtypeStruct((1, 1), jnp.float32),
            jax.ShapeDtypeStruct((_NBLK, 1, _INV_B), jnp.int32),
            jax.ShapeDtypeStruct((_NBLK, 1, _INV_B), jnp.float32),
        ],
    )(x_flat, Wr, br.reshape(1, E))


@functools.lru_cache(maxsize=None)
def _sc_mesh():
    return plsc.VectorSubcoreMesh(
        core_axis_name="c", subcore_axis_name="s",
        num_cores=NC, num_subcores=NS)


GPW = NA // NW


def _sc_gather(tok_sorted, x_flat):
    @functools.partial(
        pl.kernel,
        out_type=jax.ShapeDtypeStruct((NA, D_MODEL), jnp.float32),
        mesh=_sc_mesh(),
        scratch_types=[
            pltpu.VMEM((GPW,), jnp.int32),
            pltpu.VMEM((GPW, D_MODEL), jnp.float32),
            pltpu.SemaphoreType.DMA,
        ],
    )
    def body(tok_hbm, x_hbm, out_hbm, idx_v, rows_v, sem):
        wid = lax.axis_index("s") * NC + lax.axis_index("c")
        base = wid * GPW
        pltpu.sync_copy(tok_hbm.at[pl.ds(base, GPW)], idx_v)
        pltpu.async_copy(x_hbm.at[idx_v], rows_v, sem).wait()
        pltpu.sync_copy(rows_v, out_hbm.at[pl.ds(base, GPW)])

    return body(tok_sorted, x_flat)


def _ffn_body(meta_ref, x_ref, w1_ref, b1_ref, w2_ref, b2_ref, ws_ref,
              out_ref):
    t = pl.program_id(0)
    r = meta_ref[1, t]
    r_prev = meta_ref[1, jnp.maximum(t - 1, 0)]
    first = (t == 0) | (r != r_prev)
    lo = meta_ref[2, t]
    hi = meta_ref[3, t]

    rows = lax.broadcasted_iota(jnp.int32, (TM, 1), 0)
    valid = (rows >= lo) & (rows < hi)

    x = x_ref[...]
    h = jnp.dot(x, w1_ref[0], preferred_element_type=jnp.float32)
    h = h + b1_ref[0]
    h = h * (1.0 / (1.0 + jnp.exp(-h)))
    contrib = jnp.dot(h, w2_ref[0], preferred_element_type=jnp.float32)
    contrib = contrib + b2_ref[0]
    update = jnp.where(valid, ws_ref[...] * contrib, 0.0)

    @pl.when(first)
    def _():
        out_ref[...] = update

    @pl.when(jnp.logical_not(first))
    def _():
        out_ref[...] += update


def _ffn_grouped(meta, x_sorted, W1, b1, W2, b2, w_sorted):
    grid_spec = pltpu.PrefetchScalarGridSpec(
        num_scalar_prefetch=1,
        grid=(TPAD,),
        in_specs=[
            pl.BlockSpec((TM, D_MODEL), lambda t, m: (m[1, t], 0)),
            pl.BlockSpec((1, D_MODEL, D_FF), lambda t, m: (m[0, t], 0, 0)),
            pl.BlockSpec((1, 1, D_FF), lambda t, m: (m[0, t], 0, 0)),
            pl.BlockSpec((1, D_FF, D_MODEL), lambda t, m: (m[0, t], 0, 0)),
            pl.BlockSpec((1, 1, D_MODEL), lambda t, m: (m[0, t], 0, 0)),
            pl.BlockSpec((TM, 1), lambda t, m: (m[1, t], 0)),
        ],
        out_specs=pl.BlockSpec((TM, D_MODEL), lambda t, m: (m[1, t], 0)),
    )
    return pl.pallas_call(
        _ffn_body,
        grid_spec=grid_spec,
        out_shape=jax.ShapeDtypeStruct((NA, D_MODEL), jnp.float32),
    )(meta, x_sorted, W1, b1.reshape(E, 1, D_FF), W2,
      b2.reshape(E, 1, D_MODEL), w_sorted)


def _sc_combine(i0, i1, out_sorted):
    @functools.partial(
        pl.kernel,
        out_type=jax.ShapeDtypeStruct((N, D_MODEL), jnp.float32),
        mesh=_sc_mesh(),
        scratch_types=[
            pltpu.VMEM((CPW,), jnp.int32),
            pltpu.VMEM((CPW,), jnp.int32),
            pltpu.VMEM((CPW, D_MODEL), jnp.float32),
            pltpu.VMEM((CPW, D_MODEL), jnp.float32),
            pltpu.SemaphoreType.DMA,
        ],
    )
    def body(i0_hbm, i1_hbm, os_hbm, out_hbm, idx0, idx1, r0, r1, sem):
        wid = lax.axis_index("s") * NC + lax.axis_index("c")
        base = wid * CPW
        pltpu.sync_copy(i0_hbm.at[pl.ds(base, CPW)], idx0)
        pltpu.sync_copy(i1_hbm.at[pl.ds(base, CPW)], idx1)
        pltpu.async_copy(os_hbm.at[idx0], r0, sem).wait()
        pltpu.async_copy(os_hbm.at[idx1], r1, sem).wait()

        def add_row(i, _):
            for j in range(D_MODEL // 16):
                sl = pl.ds(j * 16, 16)
                r0[i, sl] = r0[i, sl] + r1[i, sl]
            return 0

        lax.fori_loop(0, CPW, add_row, 0)
        pltpu.sync_copy(r0, out_hbm.at[pl.ds(base, CPW)])

    return body(i0, i1, out_sorted)


@jax.jit
def kernel(x, Wr, br, W1, b1, W2, b2):
    B, L, D = x.shape
    x_flat = x.reshape(-1, D)
    wk, posi, meta, aux, tok3, ws3 = _router_inv(x_flat, Wr, br)
    tok_sorted = tok3.reshape(-1)
    w_sorted = ws3.reshape(-1, 1)
    i0 = posi[:, 0]
    i1 = posi[:, 1]
    x_sorted = _sc_gather(tok_sorted, x_flat)
    out_sorted = _ffn_grouped(meta, x_sorted, W1, b1, W2, b2, w_sorted)
    out = _sc_combine(i0, i1, out_sorted)
    return out.reshape(B, L, D), aux.reshape(1)


# speedup vs baseline: 1.2141x; 1.0508x over previous
import functools

import jax
import jax.numpy as jnp
from jax import lax
from jax.experimental import pallas as pl
from jax.experimental.pallas import tpu as pltpu
from jax.experimental.pallas import tpu_sc as plsc

D_MODEL = 768
D_FF = 3072
E = 8
TEMP = 0.7
N = 2048
NA = 2 * N

TM = 256
NB = NA // TM
TPAD = 24

NC, NS = 2, 16
NW = NC * NS
SPW = N // NW
CPW = N // NW


def _rinv_body(x_ref, wr_ref, br_ref,
               wk_ref, posi_ref, meta_ref, aux_ref, tok_ref, ws_ref):
    b = pl.program_id(0)

    @pl.when(b == 0)
    def _router_step():
        x = x_ref[...]
        logits = jnp.dot(x, wr_ref[...], preferred_element_type=jnp.float32)
        logits = logits + br_ref[...]
        z = logits / TEMP
        z = z - jnp.max(z, axis=1, keepdims=True)
        ez = jnp.exp(z)
        probs = ez / jnp.sum(ez, axis=1, keepdims=True)

        e_ids = lax.broadcasted_iota(jnp.int32, (N, E), 1)
        m1 = jnp.max(probs, axis=1, keepdims=True)
        i1 = jnp.min(jnp.where(probs == m1, e_ids, E), axis=1, keepdims=True)
        oh1 = (e_ids == i1).astype(jnp.float32)
        p2 = jnp.where(oh1 > 0, -jnp.inf, probs)
        m2 = jnp.max(p2, axis=1, keepdims=True)
        i2 = jnp.min(jnp.where(p2 == m2, e_ids, E), axis=1, keepdims=True)
        oh2 = (e_ids == i2).astype(jnp.float32)

        denom = m1 + m2 + 1e-6
        w1n = m1 / denom
        w2n = m2 / denom

        cum = oh1 + oh2
        k = 1
        while k < N:
            cum = cum + jnp.concatenate(
                [jnp.zeros((k, E), jnp.float32), cum[:N - k]], axis=0)
            k *= 2

        counts = cum[N - 1:N, :]
        offs = jnp.zeros_like(counts)
        for k in range(1, E):
            offs = offs + jnp.concatenate(
                [jnp.zeros((1, k), jnp.float32), counts[:, :E - k]], axis=1)

        slot = offs + cum - 1.0
        pos0 = jnp.sum(oh1 * slot, axis=1, keepdims=True)
        pos1 = jnp.sum(oh2 * slot, axis=1, keepdims=True)

        wk_ref[...] = jnp.concatenate([w1n, w2n], axis=1)
        posi_ref[...] = jnp.concatenate([pos0, pos1],
                                        axis=1).astype(jnp.int32)

        f_i = counts / N
        p_i = jnp.mean(probs, axis=0, keepdims=True)
        aux_ref[...] = E * jnp.sum(f_i * p_i, keepdims=True).reshape(1, 1)

        starts = offs
        ends = offs + counts
        r_col = lax.broadcasted_iota(jnp.int32, (NB, 1), 0)
        r_colf = r_col.astype(jnp.float32)
        flag = ((starts < (r_colf + 1.0) * TM) & (ends > r_colf * TM)
                & (counts > 0))
        flag_f = flag.astype(jnp.float32)
        within = flag_f
        k = 1
        while k < E:
            within = within + jnp.concatenate(
                [jnp.zeros((NB, k), jnp.float32), within[:, :E - k]], axis=1)
            k *= 2
        rowsum = jnp.sum(flag_f, axis=1, keepdims=True)
        prev = rowsum
        k = 1
        while k < NB:
            prev = prev + jnp.concatenate(
                [jnp.zeros((k, 1), jnp.float32), prev[:NB - k]], axis=0)
            k *= 2
        prev = prev - rowsum
        tidx = prev + within - 1.0
        lo = jnp.clip(starts - r_colf * TM, 0.0, float(TM))
        hi = jnp.clip(ends - r_colf * TM, 0.0, float(TM))

        trange = lax.broadcasted_iota(jnp.int32, (1, TPAD),
                                      1).astype(jnp.float32)
        me = jnp.zeros((1, TPAD), jnp.float32)
        mr = jnp.zeros((1, TPAD), jnp.float32)
        mlo = jnp.zeros((1, TPAD), jnp.float32)
        mhi = jnp.zeros((1, TPAD), jnp.float32)
        cov = jnp.zeros((1, TPAD), jnp.float32)
        for e in range(E):
            mf = ((tidx[:, e:e + 1] == trange)
                  & flag[:, e:e + 1]).astype(jnp.float32)
            me = me + jnp.sum(mf, axis=0, keepdims=True) * e
            mr = mr + jnp.sum(mf * r_colf, axis=0, keepdims=True)
            mlo = mlo + jnp.sum(mf * lo[:, e:e + 1], axis=0, keepdims=True)
            mhi = mhi + jnp.sum(mf * hi[:, e:e + 1], axis=0, keepdims=True)
            cov = cov + jnp.sum(mf, axis=0, keepdims=True)
        mr = mr + (1.0 - cov) * (NB - 1)
        meta_ref[...] = jnp.concatenate([me, mr, mlo, mhi],
                                        axis=0).astype(jnp.int32)

    @pl.when(b > 0)
    def _invert_step():
        slot = (lax.broadcasted_iota(jnp.int32, (1, _INV_B), 1)
                + _INV_B * (b - 1))
        posi = posi_ref[...]
        wkv = wk_ref[...]
        m0 = (posi[:, 0:1] == slot).astype(jnp.float32)
        m1 = (posi[:, 1:2] == slot).astype(jnp.float32)
        t_ids = lax.broadcasted_iota(jnp.int32, (N, 1),
                                     0).astype(jnp.float32)
        tok = jnp.sum((m0 + m1) * t_ids, axis=0, keepdims=True)
        w = jnp.sum(m0 * wkv[:, 0:1] + m1 * wkv[:, 1:2],
                    axis=0, keepdims=True)
        tok_ref[0] = tok.astype(jnp.int32)
        ws_ref[0] = w


_INV_B = 1024
_NBLK = NA // _INV_B


def _router_inv(x_flat, Wr, br):
    return pl.pallas_call(
        _rinv_body,
        grid=(1 + _NBLK,),
        in_specs=[
            pl.BlockSpec((N, D_MODEL), lambda b: (0, 0)),
            pl.BlockSpec((D_MODEL, E), lambda b: (0, 0)),
            pl.BlockSpec((1, E), lambda b: (0, 0)),
        ],
        out_specs=[
            pl.BlockSpec((N, 2), lambda b: (0, 0)),
            pl.BlockSpec((N, 2), lambda b: (0, 0)),
            pl.BlockSpec((4, TPAD), lambda b: (0, 0)),
            pl.BlockSpec((1, 1), lambda b: (0, 0)),
            pl.BlockSpec((1, 1, _INV_B),
                         lambda b: (jnp.maximum(b - 1, 0), 0, 0)),
            pl.BlockSpec((1, 1, _INV_B),
                         lambda b: (jnp.maximum(b - 1, 0), 0, 0)),
        ],
        out_shape=[
            jax.ShapeDtypeStruct((N, 2), jnp.float32),
            jax.ShapeDtypeStruct((N, 2), jnp.int32),
            jax.ShapeDtypeStruct((4, TPAD), jnp.int32),
            jax.ShapeDtypeStruct((1, 1), jnp.float32),
            jax.ShapeDtypeStruct((_NBLK, 1, _INV_B), jnp.int32),
            jax.ShapeDtypeStruct((_NBLK, 1, _INV_B), jnp.float32),
        ],
    )(x_flat, Wr, br.reshape(1, E))


@functools.lru_cache(maxsize=None)
def _sc_mesh():
    return plsc.VectorSubcoreMesh(
        core_axis_name="c", subcore_axis_name="s",
        num_cores=NC, num_subcores=NS)


GPW = NA // NW


def _sc_gather(tok_sorted, x_flat):
    @functools.partial(
        pl.kernel,
        out_type=jax.ShapeDtypeStruct((NA, D_MODEL), jnp.float32),
        mesh=_sc_mesh(),
        scratch_types=[
            pltpu.VMEM((GPW,), jnp.int32),
            pltpu.VMEM((GPW, D_MODEL), jnp.float32),
            pltpu.SemaphoreType.DMA,
        ],
    )
    def body(tok_hbm, x_hbm, out_hbm, idx_v, rows_v, sem):
        wid = lax.axis_index("s") * NC + lax.axis_index("c")
        base = wid * GPW
        pltpu.sync_copy(tok_hbm.at[pl.ds(base, GPW)], idx_v)
        pltpu.async_copy(x_hbm.at[idx_v], rows_v, sem).wait()
        pltpu.sync_copy(rows_v, out_hbm.at[pl.ds(base, GPW)])

    return body(tok_sorted, x_flat)


def _ffn_body(meta_ref, x_ref, w1_ref, b1_ref, w2_ref, b2_ref, ws_ref,
              out_ref):
    t = pl.program_id(0)
    r = meta_ref[1, t]
    r_prev = meta_ref[1, jnp.maximum(t - 1, 0)]
    first = (t == 0) | (r != r_prev)
    lo = meta_ref[2, t]
    hi = meta_ref[3, t]

    rows = lax.broadcasted_iota(jnp.int32, (TM, 1), 0)
    valid = (rows >= lo) & (rows < hi)

    x = x_ref[...]
    h = jnp.dot(x, w1_ref[0], preferred_element_type=jnp.float32)
    h = h + b1_ref[0]
    h = h * (1.0 / (1.0 + jnp.exp(-h)))
    contrib = jnp.dot(h, w2_ref[0], preferred_element_type=jnp.float32)
    contrib = contrib + b2_ref[0]
    update = jnp.where(valid, ws_ref[...] * contrib, 0.0)

    @pl.when(first)
    def _():
        out_ref[...] = update

    @pl.when(jnp.logical_not(first))
    def _():
        out_ref[...] += update


def _ffn_grouped(meta, x_sorted, W1, b1, W2, b2, w_sorted):
    grid_spec = pltpu.PrefetchScalarGridSpec(
        num_scalar_prefetch=1,
        grid=(TPAD,),
        in_specs=[
            pl.BlockSpec((TM, D_MODEL), lambda t, m: (m[1, t], 0)),
            pl.BlockSpec((1, D_MODEL, D_FF), lambda t, m: (m[0, t], 0, 0)),
            pl.BlockSpec((1, 1, D_FF), lambda t, m: (m[0, t], 0, 0)),
            pl.BlockSpec((1, D_FF, D_MODEL), lambda t, m: (m[0, t], 0, 0)),
            pl.BlockSpec((1, 1, D_MODEL), lambda t, m: (m[0, t], 0, 0)),
            pl.BlockSpec((TM, 1), lambda t, m: (m[1, t], 0)),
        ],
        out_specs=pl.BlockSpec((TM, D_MODEL), lambda t, m: (m[1, t], 0)),
    )
    return pl.pallas_call(
        _ffn_body,
        grid_spec=grid_spec,
        out_shape=jax.ShapeDtypeStruct((NA, D_MODEL), jnp.float32),
    )(meta, x_sorted, W1, b1.reshape(E, 1, D_FF), W2,
      b2.reshape(E, 1, D_MODEL), w_sorted)


def _sc_combine(i0, i1, out_sorted):
    @functools.partial(
        pl.kernel,
        out_type=jax.ShapeDtypeStruct((N, D_MODEL), jnp.float32),
        mesh=_sc_mesh(),
        scratch_types=[
            pltpu.VMEM((CPW,), jnp.int32),
            pltpu.VMEM((CPW,), jnp.int32),
            pltpu.VMEM((CPW, D_MODEL), jnp.float32),
            pltpu.VMEM((CPW, D_MODEL), jnp.float32),
            pltpu.SemaphoreType.DMA,
        ],
    )
    def body(i0_hbm, i1_hbm, os_hbm, out_hbm, idx0, idx1, r0, r1, sem):
        wid = lax.axis_index("s") * NC + lax.axis_index("c")
        base = wid * CPW
        pltpu.sync_copy(i0_hbm.at[pl.ds(base, CPW)], idx0)
        pltpu.sync_copy(i1_hbm.at[pl.ds(base, CPW)], idx1)
        pltpu.async_copy(os_hbm.at[idx0], r0, sem).wait()
        pltpu.async_copy(os_hbm.at[idx1], r1, sem).wait()

        def add_row(i, _):
            for j in range(D_MODEL // 16):
                sl = pl.ds(j * 16, 16)
                r0[i, sl] = r0[i, sl] + r1[i, sl]
            return 0

        lax.fori_loop(0, CPW, add_row, 0)
        pltpu.sync_copy(r0, out_hbm.at[pl.ds(base, CPW)])

    return body(i0, i1, out_sorted)


@jax.jit
def kernel(x, Wr, br, W1, b1, W2, b2):
    B, L, D = x.shape
    x_flat = x.reshape(-1, D)
    wk, posi, meta, aux, tok3, ws3 = _router_inv(x_flat, Wr, br)
    tok_sorted = tok3.reshape(-1)
    w_sorted = ws3.reshape(-1, 1)
    i0 = posi[:, 0]
    i1 = posi[:, 1]
    x_sorted = _sc_gather(tok_sorted, x_flat)
    out_sorted = _ffn_grouped(meta, x_sorted, W1, b1, W2, b2, w_sorted)
    out = _sc_combine(i0, i1, out_sorted)
    return out.reshape(B, L, D), aux.reshape(1)


# speedup vs baseline: 1.2159x; 1.0015x over previous
import functools

import jax
import jax.numpy as jnp
from jax import lax
from jax.experimental import pallas as pl
from jax.experimental.pallas import tpu as pltpu
from jax.experimental.pallas import tpu_sc as plsc

D_MODEL = 768
D_FF = 3072
E = 8
TEMP = 0.7
N = 2048
NA = 2 * N

TM = 256
NB = NA // TM
TPAD = 24

NC, NS = 2, 16
NW = NC * NS
CPW = N // NW


def _rinv_body(x_ref, wr_ref, br_ref,
               wk_ref, posi_ref, meta_ref, aux_ref, tok_ref, ws_ref):
    b = pl.program_id(0)

    @pl.when(b == 0)
    def _router_step():
        x = x_ref[...]
        logits = jnp.dot(x, wr_ref[...], preferred_element_type=jnp.float32)
        logits = logits + br_ref[...]
        z = logits / TEMP
        z = z - jnp.max(z, axis=1, keepdims=True)
        ez = jnp.exp(z)
        probs = ez / jnp.sum(ez, axis=1, keepdims=True)

        e_ids = lax.broadcasted_iota(jnp.int32, (N, E), 1)
        m1 = jnp.max(probs, axis=1, keepdims=True)
        i1 = jnp.min(jnp.where(probs == m1, e_ids, E), axis=1, keepdims=True)
        oh1 = (e_ids == i1).astype(jnp.float32)
        p2 = jnp.where(oh1 > 0, -jnp.inf, probs)
        m2 = jnp.max(p2, axis=1, keepdims=True)
        i2 = jnp.min(jnp.where(p2 == m2, e_ids, E), axis=1, keepdims=True)
        oh2 = (e_ids == i2).astype(jnp.float32)

        denom = m1 + m2 + 1e-6
        w1n = m1 / denom
        w2n = m2 / denom

        cum = oh1 + oh2
        k = 1
        while k < N:
            cum = cum + jnp.concatenate(
                [jnp.zeros((k, E), jnp.float32), cum[:N - k]], axis=0)
            k *= 2

        counts = cum[N - 1:N, :]
        offs = jnp.zeros_like(counts)
        for k in range(1, E):
            offs = offs + jnp.concatenate(
                [jnp.zeros((1, k), jnp.float32), counts[:, :E - k]], axis=1)

        slot = offs + cum - 1.0
        pos0 = jnp.sum(oh1 * slot, axis=1, keepdims=True)
        pos1 = jnp.sum(oh2 * slot, axis=1, keepdims=True)

        wk_ref[...] = jnp.concatenate([w1n, w2n], axis=1)
        posi_ref[...] = jnp.concatenate([pos0, pos1],
                                        axis=1).astype(jnp.int32)

        f_i = counts / N
        p_i = jnp.mean(probs, axis=0, keepdims=True)
        aux_ref[...] = E * jnp.sum(f_i * p_i, keepdims=True).reshape(1, 1)

        starts = offs
        ends = offs + counts
        r_col = lax.broadcasted_iota(jnp.int32, (NB, 1), 0)
        r_colf = r_col.astype(jnp.float32)
        flag = ((starts < (r_colf + 1.0) * TM) & (ends > r_colf * TM)
                & (counts > 0))
        flag_f = flag.astype(jnp.float32)
        within = flag_f
        k = 1
        while k < E:
            within = within + jnp.concatenate(
                [jnp.zeros((NB, k), jnp.float32), within[:, :E - k]], axis=1)
            k *= 2
        rowsum = jnp.sum(flag_f, axis=1, keepdims=True)
        prev = rowsum
        k = 1
        while k < NB:
            prev = prev + jnp.concatenate(
                [jnp.zeros((k, 1), jnp.float32), prev[:NB - k]], axis=0)
            k *= 2
        prev = prev - rowsum
        tidx = prev + within - 1.0
        lo = jnp.clip(starts - r_colf * TM, 0.0, float(TM))
        hi = jnp.clip(ends - r_colf * TM, 0.0, float(TM))

        trange = lax.broadcasted_iota(jnp.int32, (1, TPAD),
                                      1).astype(jnp.float32)
        me = jnp.zeros((1, TPAD), jnp.float32)
        mr = jnp.zeros((1, TPAD), jnp.float32)
        mlo = jnp.zeros((1, TPAD), jnp.float32)
        mhi = jnp.zeros((1, TPAD), jnp.float32)
        cov = jnp.zeros((1, TPAD), jnp.float32)
        for e in range(E):
            mf = ((tidx[:, e:e + 1] == trange)
                  & flag[:, e:e + 1]).astype(jnp.float32)
            me = me + jnp.sum(mf, axis=0, keepdims=True) * e
            mr = mr + jnp.sum(mf * r_colf, axis=0, keepdims=True)
            mlo = mlo + jnp.sum(mf * lo[:, e:e + 1], axis=0, keepdims=True)
            mhi = mhi + jnp.sum(mf * hi[:, e:e + 1], axis=0, keepdims=True)
            cov = cov + jnp.sum(mf, axis=0, keepdims=True)
        mr = mr + (1.0 - cov) * (NB - 1)
        meta_ref[...] = jnp.concatenate([me, mr, mlo, mhi],
                                        axis=0).astype(jnp.int32)

    @pl.when(b > 0)
    def _invert_step():
        slot = (lax.broadcasted_iota(jnp.int32, (1, _INV_B), 1)
                + _INV_B * (b - 1))
        posi = posi_ref[...]
        wkv = wk_ref[...]
        m0 = (posi[:, 0:1] == slot).astype(jnp.float32)
        m1 = (posi[:, 1:2] == slot).astype(jnp.float32)
        t_ids = lax.broadcasted_iota(jnp.int32, (N, 1),
                                     0).astype(jnp.float32)
        tok = jnp.sum((m0 + m1) * t_ids, axis=0, keepdims=True)
        w = jnp.sum(m0 * wkv[:, 0:1] + m1 * wkv[:, 1:2],
                    axis=0, keepdims=True)
        tok_ref[0] = tok.astype(jnp.int32)
        ws_ref[0] = w


_INV_B = 1024
_NBLK = NA // _INV_B


def _router_inv(x_flat, Wr, br):
    return pl.pallas_call(
        _rinv_body,
        grid=(1 + _NBLK,),
        in_specs=[
            pl.BlockSpec((N, D_MODEL), lambda b: (0, 0)),
            pl.BlockSpec((D_MODEL, E), lambda b: (0, 0)),
            pl.BlockSpec((1, E), lambda b: (0, 0)),
        ],
        out_specs=[
            pl.BlockSpec((N, 2), lambda b: (0, 0)),
            pl.BlockSpec((N, 2), lambda b: (0, 0)),
            pl.BlockSpec((4, TPAD), lambda b: (0, 0)),
            pl.BlockSpec((1, 1), lambda b: (0, 0)),
            pl.BlockSpec((1, 1, _INV_B),
                         lambda b: (jnp.maximum(b - 1, 0), 0, 0)),
            pl.BlockSpec((1, 1, _INV_B),
                         lambda b: (jnp.maximum(b - 1, 0), 0, 0)),
        ],
        out_shape=[
            jax.ShapeDtypeStruct((N, 2), jnp.float32),
            jax.ShapeDtypeStruct((N, 2), jnp.int32),
            jax.ShapeDtypeStruct((4, TPAD), jnp.int32),
            jax.ShapeDtypeStruct((1, 1), jnp.float32),
            jax.ShapeDtypeStruct((_NBLK, 1, _INV_B), jnp.int32),
            jax.ShapeDtypeStruct((_NBLK, 1, _INV_B), jnp.float32),
        ],
    )(x_flat, Wr, br.reshape(1, E))


@functools.lru_cache(maxsize=None)
def _sc_mesh():
    return plsc.VectorSubcoreMesh(
        core_axis_name="c", subcore_axis_name="s",
        num_cores=NC, num_subcores=NS)


GPW = NA // NW


def _sc_gather(tok_sorted, x_flat):
    @functools.partial(
        pl.kernel,
        out_type=jax.ShapeDtypeStruct((NA, D_MODEL), jnp.float32),
        mesh=_sc_mesh(),
        scratch_types=[
            pltpu.VMEM((GPW,), jnp.int32),
            pltpu.VMEM((GPW, D_MODEL), jnp.float32),
            pltpu.SemaphoreType.DMA,
        ],
    )
    def body(tok_hbm, x_hbm, out_hbm, idx_v, rows_v, sem):
        wid = lax.axis_index("s") * NC + lax.axis_index("c")
        base = wid * GPW
        pltpu.sync_copy(tok_hbm.at[pl.ds(base, GPW)], idx_v)
        pltpu.async_copy(x_hbm.at[idx_v], rows_v, sem).wait()
        pltpu.sync_copy(rows_v, out_hbm.at[pl.ds(base, GPW)])

    return body(tok_sorted, x_flat)


def _ffn_body(meta_ref, x_ref, w1_ref, b1_ref, w2_ref, b2_ref, ws_ref,
              out_ref):
    t = pl.program_id(0)
    r = meta_ref[1, t]
    r_prev = meta_ref[1, jnp.maximum(t - 1, 0)]
    first = (t == 0) | (r != r_prev)
    lo = meta_ref[2, t]
    hi = meta_ref[3, t]

    rows = lax.broadcasted_iota(jnp.int32, (TM, 1), 0)
    valid = (rows >= lo) & (rows < hi)

    x = x_ref[...]
    h = jnp.dot(x, w1_ref[0], preferred_element_type=jnp.float32)
    h = h + b1_ref[0]
    h = h * (1.0 / (1.0 + jnp.exp(-h)))
    contrib = jnp.dot(h, w2_ref[0], preferred_element_type=jnp.float32)
    contrib = contrib + b2_ref[0]
    update = jnp.where(valid, ws_ref[...] * contrib, 0.0)

    @pl.when(first)
    def _():
        out_ref[...] = update

    @pl.when(jnp.logical_not(first))
    def _():
        out_ref[...] += update


def _ffn_grouped(meta, x_sorted, W1, b1, W2, b2, w_sorted):
    grid_spec = pltpu.PrefetchScalarGridSpec(
        num_scalar_prefetch=1,
        grid=(TPAD,),
        in_specs=[
            pl.BlockSpec((TM, D_MODEL), lambda t, m: (m[1, t], 0)),
            pl.BlockSpec((1, D_MODEL, D_FF), lambda t, m: (m[0, t], 0, 0)),
            pl.BlockSpec((1, 1, D_FF), lambda t, m: (m[0, t], 0, 0)),
            pl.BlockSpec((1, D_FF, D_MODEL), lambda t, m: (m[0, t], 0, 0)),
            pl.BlockSpec((1, 1, D_MODEL), lambda t, m: (m[0, t], 0, 0)),
            pl.BlockSpec((TM, 1), lambda t, m: (m[1, t], 0)),
        ],
        out_specs=pl.BlockSpec((TM, D_MODEL), lambda t, m: (m[1, t], 0)),
    )
    return pl.pallas_call(
        _ffn_body,
        grid_spec=grid_spec,
        out_shape=jax.ShapeDtypeStruct((NA, D_MODEL), jnp.float32),
    )(meta, x_sorted, W1, b1.reshape(E, 1, D_FF), W2,
      b2.reshape(E, 1, D_MODEL), w_sorted)


def _sc_combine(i0, i1, out_sorted):
    @functools.partial(
        pl.kernel,
        out_type=jax.ShapeDtypeStruct((N, D_MODEL), jnp.float32),
        mesh=_sc_mesh(),
        scratch_types=[
            pltpu.VMEM((CPW,), jnp.int32),
            pltpu.VMEM((CPW,), jnp.int32),
            pltpu.VMEM((CPW, D_MODEL), jnp.float32),
            pltpu.VMEM((CPW, D_MODEL), jnp.float32),
            pltpu.SemaphoreType.DMA,
        ],
    )
    def body(i0_hbm, i1_hbm, os_hbm, out_hbm, idx0, idx1, r0, r1, sem):
        wid = lax.axis_index("s") * NC + lax.axis_index("c")
        base = wid * CPW
        pltpu.sync_copy(i0_hbm.at[pl.ds(base, CPW)], idx0)
        pltpu.sync_copy(i1_hbm.at[pl.ds(base, CPW)], idx1)
        pltpu.async_copy(os_hbm.at[idx0], r0, sem).wait()
        pltpu.async_copy(os_hbm.at[idx1], r1, sem).wait()

        def add_row(i, _):
            for j in range(D_MODEL // 16):
                sl = pl.ds(j * 16, 16)
                r0[i, sl] = r0[i, sl] + r1[i, sl]
            return 0

        lax.fori_loop(0, CPW, add_row, 0)
        pltpu.sync_copy(r0, out_hbm.at[pl.ds(base, CPW)])

    return body(i0, i1, out_sorted)


@jax.jit
def kernel(x, Wr, br, W1, b1, W2, b2):
    B, L, D = x.shape
    x_flat = x.reshape(-1, D)
    wk, posi, meta, aux, tok3, ws3 = _router_inv(x_flat, Wr, br)
    tok_sorted = tok3.reshape(-1)
    w_sorted = ws3.reshape(-1, 1)
    i0 = posi[:, 0]
    i1 = posi[:, 1]
    x_sorted = _sc_gather(tok_sorted, x_flat)
    out_sorted = _ffn_grouped(meta, x_sorted, W1, b1, W2, b2, w_sorted)
    out = _sc_combine(i0, i1, out_sorted)
    return out.reshape(B, L, D), aux.reshape(1)
